# baseline (speedup 1.0000x reference)
import functools

import jax
import jax.numpy as jnp
from jax import lax
from jax.experimental import pallas as pl
from jax.experimental.pallas import tpu as pltpu
from jax.experimental.pallas import tpu_sc as plsc

E = 8
TOPK = 2
D = 768
DFF = 2048
T = 2048
B = 1024
NBLK = (TOPK * T) // B + E
P = NBLK * B
NC = 2
NS = 16
NW = NC * NS
TPW = T // NW
NG = 4

_mesh = plsc.VectorSubcoreMesh(core_axis_name="c", subcore_axis_name="s")


def _route_body(lg_ref, rn_ref, pos0_ref, pos1_ref, w0_ref, w1_ref, be_ref):
    lane = lax.broadcasted_iota(jnp.int32, (T, 128), 1)
    lane8 = lax.broadcasted_iota(jnp.int32, (T, E), 1)
    lg = lg_ref[...]

    m1 = jnp.max(lg, axis=1, keepdims=True)
    e0 = jnp.min(jnp.where(lg == m1, lane8, 999), axis=1, keepdims=True)
    lg2 = jnp.where(lane8 == e0, jnp.float32(-3e38), lg)
    m2 = jnp.max(lg2, axis=1, keepdims=True)
    e1 = jnp.min(jnp.where(lg2 == m2, lane8, 999), axis=1, keepdims=True)

    d12 = jnp.exp(m2 - m1)
    w1n = 1.0 / (1.0 + d12)
    w2n = d12 * w1n
    den8 = jnp.sum(jnp.exp(lg - m1), axis=1, keepdims=True)
    p1 = 1.0 / den8
    p2 = d12 / den8
    rn = rn_ref[0:1, 0:1] > 0.5
    wa = jnp.where(rn, w1n, p1)
    wb = jnp.where(rn, w2n, p2)

    oh0 = (lane == e0).astype(jnp.float32)
    oh1 = (lane == e1).astype(jnp.float32)
    oh = oh0 + oh1

    r_i = lax.broadcasted_iota(jnp.int32, (T, T), 0)
    c_i = lax.broadcasted_iota(jnp.int32, (T, T), 1)
    ltri = (c_i < r_i).astype(jnp.float32)
    pre = lax.dot_general(ltri, oh, (((1,), (0,)), ((), ())),
                          preferred_element_type=jnp.float32)

    counts = pre[T - 1:T, :] + oh[T - 1:T, :]
    nblk = jnp.floor((counts + float(B - 1)) * (1.0 / B))
    u_r = lax.broadcasted_iota(jnp.int32, (128, 128), 0)
    u_c = lax.broadcasted_iota(jnp.int32, (128, 128), 1)
    utri = (u_r < u_c).astype(jnp.float32)
    blkbase = lax.dot_general(nblk, utri, (((1,), (0,)), ((), ())),
                              preferred_element_type=jnp.float32)
    rowbase = blkbase * float(B)

    slot = rowbase + pre
    pos0 = jnp.sum(jnp.where(lane == e0, slot, 0.0), axis=1, keepdims=True)
    pos1 = jnp.sum(jnp.where(lane == e1, slot, 0.0), axis=1, keepdims=True)
    pos0_ref[...] = jnp.reshape(pos0.astype(jnp.int32), (T // 128, 128))
    pos1_ref[...] = jnp.reshape(pos1.astype(jnp.int32), (T // 128, 128))
    w0_ref[...] = jnp.broadcast_to(wa, (T, 128))
    w1_ref[...] = jnp.broadcast_to(wb, (T, 128))

    bvec = lax.broadcasted_iota(jnp.int32, (8, 128), 1).astype(jnp.float32)
    becnt = jnp.zeros((8, 128), jnp.int32)
    for e in range(E):
        becnt = becnt + (bvec >= blkbase[0:1, e:e + 1]).astype(jnp.int32)
    be_ref[...] = jnp.maximum(becnt - 1, 0)


def _route(logits, rn2d):
    return pl.pallas_call(
        _route_body,
        grid=(1,),
        in_specs=[
            pl.BlockSpec((T, E), lambda i: (0, 0)),
            pl.BlockSpec((8, 128), lambda i: (0, 0)),
        ],
        out_specs=[
            pl.BlockSpec((T // 128, 128), lambda i: (0, 0)),
            pl.BlockSpec((T // 128, 128), lambda i: (0, 0)),
            pl.BlockSpec((T, 128), lambda i: (0, 0)),
            pl.BlockSpec((T, 128), lambda i: (0, 0)),
            pl.BlockSpec((8, 128), lambda i: (0, 0)),
        ],
        out_shape=[
            jax.ShapeDtypeStruct((T // 128, 128), jnp.int32),
            jax.ShapeDtypeStruct((T // 128, 128), jnp.int32),
            jax.ShapeDtypeStruct((T, 128), jnp.float32),
            jax.ShapeDtypeStruct((T, 128), jnp.float32),
            jax.ShapeDtypeStruct((8, 128), jnp.int32),
        ],
    )(logits, rn2d)


@functools.partial(
    pl.kernel,
    out_type=jax.ShapeDtypeStruct((P, D), jnp.float32),
    mesh=_mesh,
    scratch_types=[
        pltpu.VMEM((TPW,), jnp.int32),
        pltpu.VMEM((TPW,), jnp.int32),
        pltpu.VMEM((TPW, D), jnp.float32),
        pltpu.SemaphoreType.DMA,
        pltpu.SemaphoreType.DMA,
    ],
)
def _dispatch(x_hbm, pos0_hbm, pos1_hbm, xg_hbm,
              i0_v, i1_v, rows_v, semA, semB):
    c = lax.axis_index("c")
    s = lax.axis_index("s")
    wid = s * NC + c
    base = wid * TPW
    pltpu.sync_copy(x_hbm.at[pl.ds(base, TPW)], rows_v)
    pltpu.sync_copy(pos0_hbm.at[pl.ds(base, TPW)], i0_v)
    pltpu.sync_copy(pos1_hbm.at[pl.ds(base, TPW)], i1_v)
    cpA = pltpu.async_copy(rows_v, xg_hbm.at[i0_v], semA)
    cpB = pltpu.async_copy(rows_v, xg_hbm.at[i1_v], semB)
    cpA.wait()
    cpB.wait()


def _clampE(i):
    return jnp.minimum(jnp.maximum(i, 0), E - 1)


def _gemm_body(be_ref, x_ref, *refs):
    y_ref = refs[-1]
    ws = refs[:-1]
    ng = len(ws) // 3
    x = x_ref[...]
    p = None
    for q in range(ng):
        gq = ws[q]
        uq = ws[ng + q]
        w2q = ws[2 * ng + q]
        h1 = lax.dot_general(x, gq[0][0], (((1,), (1,)), ((), ())),
                             preferred_element_type=jnp.float32)
        h2 = lax.dot_general(x, uq[0][0], (((1,), (1,)), ((), ())),
                             preferred_element_type=jnp.float32)
        a = h1 * jax.nn.sigmoid(h1) * h2
        pq = lax.dot_general(a, w2q[0], (((1,), (1,)), ((), ())),
                             preferred_element_type=jnp.float32)
        p = pq if p is None else p + pq
    y_ref[...] = p


def _gemm(be, xg, W13, W2):
    grid_spec = pltpu.PrefetchScalarGridSpec(
        num_scalar_prefetch=1,
        grid=(NBLK,),
        in_specs=(
            [pl.BlockSpec((B, D), lambda b, be_ref: (b, 0))]
            + [pl.BlockSpec(
                   (1, 1, DFF // NG, D),
                   functools.partial(
                       lambda q, b, be_ref: (_clampE(be_ref[b]), q, 0, 0), q))
               for q in range(NG)]
            + [pl.BlockSpec(
                   (1, 1, DFF // NG, D),
                   functools.partial(
                       lambda q, b, be_ref: (_clampE(be_ref[b]), NG + q, 0, 0),
                       q))
               for q in range(NG)]
            + [pl.BlockSpec(
                   (1, D, DFF // NG),
                   functools.partial(
                       lambda q, b, be_ref: (_clampE(be_ref[b]), 0, q), q))
               for q in range(NG)]
        ),
        out_specs=pl.BlockSpec((B, D), lambda b, be_ref: (b, 0)),
    )
    w13_q = W13.reshape(E, 2 * NG, DFF // NG, D)
    return pl.pallas_call(
        _gemm_body,
        grid_spec=grid_spec,
        out_shape=jax.ShapeDtypeStruct((P, D), jnp.float32),
        compiler_params=pltpu.CompilerParams(
            dimension_semantics=("arbitrary",),
        ),
    )(be, xg, *([w13_q] * (2 * NG)), *([W2] * NG))


@functools.partial(
    pl.kernel,
    out_type=jax.ShapeDtypeStruct((T, D), jnp.float32),
    mesh=_mesh,
    scratch_types=[
        pltpu.VMEM((TPW,), jnp.int32),
        pltpu.VMEM((TPW,), jnp.int32),
        pltpu.VMEM((TPW, 128), jnp.float32),
        pltpu.VMEM((TPW, 128), jnp.float32),
        pltpu.VMEM((TPW, D), jnp.float32),
        pltpu.VMEM((TPW, D), jnp.float32),
        pltpu.SemaphoreType.DMA,
        pltpu.SemaphoreType.DMA,
        pltpu.SemaphoreType.DMA,
        pltpu.SemaphoreType.DMA,
        pltpu.SemaphoreType.DMA,
    ],
)
def _combine(pos0_hbm, pos1_hbm, w0x_hbm, w1x_hbm, y_hbm, out_hbm,
             i0_v, i1_v, w0x_v, w1x_v, ra_v, rb_v,
             semA, semB, semA2, semB2, semO):
    c = lax.axis_index("c")
    s = lax.axis_index("s")
    wid = s * NC + c
    base = wid * TPW
    H = TPW // 2
    pltpu.sync_copy(pos0_hbm.at[pl.ds(base, TPW)], i0_v)
    pltpu.sync_copy(pos1_hbm.at[pl.ds(base, TPW)], i1_v)
    cpA1 = pltpu.async_copy(y_hbm.at[i0_v.at[pl.ds(0, H)]],
                            ra_v.at[pl.ds(0, H)], semA)
    cpB1 = pltpu.async_copy(y_hbm.at[i1_v.at[pl.ds(0, H)]],
                            rb_v.at[pl.ds(0, H)], semB)
    cpA2 = pltpu.async_copy(y_hbm.at[i0_v.at[pl.ds(H, H)]],
                            ra_v.at[pl.ds(H, H)], semA2)
    cpB2 = pltpu.async_copy(y_hbm.at[i1_v.at[pl.ds(H, H)]],
                            rb_v.at[pl.ds(H, H)], semB2)
    pltpu.sync_copy(w0x_hbm.at[pl.ds(base, TPW)], w0x_v)
    pltpu.sync_copy(w1x_hbm.at[pl.ds(base, TPW)], w1x_v)
    cpA1.wait()
    cpB1.wait()

    def add_lo(j, _):
        sl = pl.ds(j * 16, 16)
        for r in range(H):
            wa = w0x_v[r, pl.ds(0, 16)]
            wb = w1x_v[r, pl.ds(0, 16)]
            ra_v[r, sl] = wa * ra_v[r, sl] + wb * rb_v[r, sl]
        return 0
    lax.fori_loop(0, D // 16, add_lo, 0)
    cpO = pltpu.async_copy(ra_v.at[pl.ds(0, H)],
                           out_hbm.at[pl.ds(base, H)], semO)
    cpA2.wait()
    cpB2.wait()

    def add_hi(j, _):
        sl = pl.ds(j * 16, 16)
        for r in range(H, TPW):
            wa = w0x_v[r, pl.ds(0, 16)]
            wb = w1x_v[r, pl.ds(0, 16)]
            ra_v[r, sl] = wa * ra_v[r, sl] + wb * rb_v[r, sl]
        return 0
    lax.fori_loop(0, D // 16, add_hi, 0)
    cpO.wait()
    pltpu.sync_copy(ra_v.at[pl.ds(H, H)], out_hbm.at[pl.ds(base + H, H)])


def kernel(hidden_states, use_grouped_topk, top_k, router_logits, renormalize, W13, W2):
    logits = router_logits.astype(jnp.float32)
    rn2d = jnp.broadcast_to(
        jnp.asarray(renormalize, jnp.float32)[None, None], (8, 128))
    pos0x, pos1x, w0x, w1x, be8 = _route(logits, rn2d)
    pos0 = pos0x.reshape(T)
    pos1 = pos1x.reshape(T)
    be = be8[0, :NBLK]
    xg = _dispatch(hidden_states, pos0, pos1)
    y = _gemm(be, xg, W13, W2)
    return _combine(pos0, pos1, w0x, w1x, y)


# speedup vs baseline: 1.1639x; 1.1639x over previous
import functools

import jax
import jax.numpy as jnp
from jax import lax
from jax.experimental import pallas as pl
from jax.experimental.pallas import tpu as pltpu
from jax.experimental.pallas import tpu_sc as plsc

E = 8
TOPK = 2
D = 768
DFF = 2048
T = 2048
B = 384
NBLK = -(-(TOPK * T) // B) + E
P = NBLK * B
NC = 2
NS = 16
NW = NC * NS
TPW = T // NW
NG = 4

_mesh = plsc.VectorSubcoreMesh(core_axis_name="c", subcore_axis_name="s")


def _route_body(lg_ref, rn_ref, pos0_ref, pos1_ref, w0_ref, w1_ref, be_ref):
    lane = lax.broadcasted_iota(jnp.int32, (T, 128), 1)
    lane8 = lax.broadcasted_iota(jnp.int32, (T, E), 1)
    lg = lg_ref[...]

    m1 = jnp.max(lg, axis=1, keepdims=True)
    e0 = jnp.min(jnp.where(lg == m1, lane8, 999), axis=1, keepdims=True)
    lg2 = jnp.where(lane8 == e0, jnp.float32(-3e38), lg)
    m2 = jnp.max(lg2, axis=1, keepdims=True)
    e1 = jnp.min(jnp.where(lg2 == m2, lane8, 999), axis=1, keepdims=True)

    d12 = jnp.exp(m2 - m1)
    w1n = 1.0 / (1.0 + d12)
    w2n = d12 * w1n
    den8 = jnp.sum(jnp.exp(lg - m1), axis=1, keepdims=True)
    p1 = 1.0 / den8
    p2 = d12 / den8
    rn = rn_ref[0:1, 0:1] > 0.5
    wa = jnp.where(rn, w1n, p1)
    wb = jnp.where(rn, w2n, p2)

    oh0 = (lane == e0).astype(jnp.float32)
    oh1 = (lane == e1).astype(jnp.float32)
    oh = oh0 + oh1

    r_i = lax.broadcasted_iota(jnp.int32, (T, T), 0)
    c_i = lax.broadcasted_iota(jnp.int32, (T, T), 1)
    ltri = (c_i < r_i).astype(jnp.float32)
    pre = lax.dot_general(ltri, oh, (((1,), (0,)), ((), ())),
                          preferred_element_type=jnp.float32)

    counts = pre[T - 1:T, :] + oh[T - 1:T, :]
    nblk = jnp.floor((counts + float(B - 1)) * (1.0 / B))
    u_r = lax.broadcasted_iota(jnp.int32, (128, 128), 0)
    u_c = lax.broadcasted_iota(jnp.int32, (128, 128), 1)
    utri = (u_r < u_c).astype(jnp.float32)
    blkbase = lax.dot_general(nblk, utri, (((1,), (0,)), ((), ())),
                              preferred_element_type=jnp.float32)
    rowbase = blkbase * float(B)

    slot = rowbase + pre
    pos0 = jnp.sum(jnp.where(lane == e0, slot, 0.0), axis=1, keepdims=True)
    pos1 = jnp.sum(jnp.where(lane == e1, slot, 0.0), axis=1, keepdims=True)
    pos0_ref[...] = jnp.reshape(pos0.astype(jnp.int32), (T // 128, 128))
    pos1_ref[...] = jnp.reshape(pos1.astype(jnp.int32), (T // 128, 128))
    w0_ref[...] = jnp.broadcast_to(wa, (T, 128))
    w1_ref[...] = jnp.broadcast_to(wb, (T, 128))

    bvec = lax.broadcasted_iota(jnp.int32, (8, 128), 1).astype(jnp.float32)
    becnt = jnp.zeros((8, 128), jnp.int32)
    for e in range(E):
        becnt = becnt + (bvec >= blkbase[0:1, e:e + 1]).astype(jnp.int32)
    be_ref[...] = jnp.maximum(becnt - 1, 0)


def _route(logits, rn2d):
    return pl.pallas_call(
        _route_body,
        grid=(1,),
        in_specs=[
            pl.BlockSpec((T, E), lambda i: (0, 0)),
            pl.BlockSpec((8, 128), lambda i: (0, 0)),
        ],
        out_specs=[
            pl.BlockSpec((T // 128, 128), lambda i: (0, 0)),
            pl.BlockSpec((T // 128, 128), lambda i: (0, 0)),
            pl.BlockSpec((T, 128), lambda i: (0, 0)),
            pl.BlockSpec((T, 128), lambda i: (0, 0)),
            pl.BlockSpec((8, 128), lambda i: (0, 0)),
        ],
        out_shape=[
            jax.ShapeDtypeStruct((T // 128, 128), jnp.int32),
            jax.ShapeDtypeStruct((T // 128, 128), jnp.int32),
            jax.ShapeDtypeStruct((T, 128), jnp.float32),
            jax.ShapeDtypeStruct((T, 128), jnp.float32),
            jax.ShapeDtypeStruct((8, 128), jnp.int32),
        ],
    )(logits, rn2d)


@functools.partial(
    pl.kernel,
    out_type=jax.ShapeDtypeStruct((P, D), jnp.float32),
    mesh=_mesh,
    scratch_types=[
        pltpu.VMEM((TPW,), jnp.int32),
        pltpu.VMEM((TPW,), jnp.int32),
        pltpu.VMEM((TPW, D), jnp.float32),
        pltpu.SemaphoreType.DMA,
        pltpu.SemaphoreType.DMA,
    ],
)
def _dispatch(x_hbm, pos0_hbm, pos1_hbm, xg_hbm,
              i0_v, i1_v, rows_v, semA, semB):
    c = lax.axis_index("c")
    s = lax.axis_index("s")
    wid = s * NC + c
    base = wid * TPW
    pltpu.sync_copy(x_hbm.at[pl.ds(base, TPW)], rows_v)
    pltpu.sync_copy(pos0_hbm.at[pl.ds(base, TPW)], i0_v)
    pltpu.sync_copy(pos1_hbm.at[pl.ds(base, TPW)], i1_v)
    cpA = pltpu.async_copy(rows_v, xg_hbm.at[i0_v], semA)
    cpB = pltpu.async_copy(rows_v, xg_hbm.at[i1_v], semB)
    cpA.wait()
    cpB.wait()


def _clampE(i):
    return jnp.minimum(jnp.maximum(i, 0), E - 1)


def _gemm_body(be_ref, x_ref, *refs):
    y_ref = refs[-1]
    ws = refs[:-1]
    ng = len(ws) // 3
    x = x_ref[...]
    p = None
    for q in range(ng):
        gq = ws[q]
        uq = ws[ng + q]
        w2q = ws[2 * ng + q]
        h1 = lax.dot_general(x, gq[0][0], (((1,), (1,)), ((), ())),
                             preferred_element_type=jnp.float32)
        h2 = lax.dot_general(x, uq[0][0], (((1,), (1,)), ((), ())),
                             preferred_element_type=jnp.float32)
        a = h1 * jax.nn.sigmoid(h1) * h2
        pq = lax.dot_general(a, w2q[0], (((1,), (1,)), ((), ())),
                             preferred_element_type=jnp.float32)
        p = pq if p is None else p + pq
    y_ref[...] = p


def _gemm(be, xg, W13, W2):
    grid_spec = pltpu.PrefetchScalarGridSpec(
        num_scalar_prefetch=1,
        grid=(NBLK,),
        in_specs=(
            [pl.BlockSpec((B, D), lambda b, be_ref: (b, 0))]
            + [pl.BlockSpec(
                   (1, 1, DFF // NG, D),
                   functools.partial(
                       lambda q, b, be_ref: (_clampE(be_ref[b]), q, 0, 0), q))
               for q in range(NG)]
            + [pl.BlockSpec(
                   (1, 1, DFF // NG, D),
                   functools.partial(
                       lambda q, b, be_ref: (_clampE(be_ref[b]), NG + q, 0, 0),
                       q))
               for q in range(NG)]
            + [pl.BlockSpec(
                   (1, D, DFF // NG),
                   functools.partial(
                       lambda q, b, be_ref: (_clampE(be_ref[b]), 0, q), q))
               for q in range(NG)]
        ),
        out_specs=pl.BlockSpec((B, D), lambda b, be_ref: (b, 0)),
    )
    w13_q = W13.reshape(E, 2 * NG, DFF // NG, D)
    return pl.pallas_call(
        _gemm_body,
        grid_spec=grid_spec,
        out_shape=jax.ShapeDtypeStruct((P, D), jnp.float32),
        compiler_params=pltpu.CompilerParams(
            dimension_semantics=("arbitrary",),
        ),
    )(be, xg, *([w13_q] * (2 * NG)), *([W2] * NG))


@functools.partial(
    pl.kernel,
    out_type=jax.ShapeDtypeStruct((T, D), jnp.float32),
    mesh=_mesh,
    scratch_types=[
        pltpu.VMEM((TPW,), jnp.int32),
        pltpu.VMEM((TPW,), jnp.int32),
        pltpu.VMEM((TPW, 128), jnp.float32),
        pltpu.VMEM((TPW, 128), jnp.float32),
        pltpu.VMEM((TPW, D), jnp.float32),
        pltpu.VMEM((TPW, D), jnp.float32),
        pltpu.SemaphoreType.DMA,
        pltpu.SemaphoreType.DMA,
        pltpu.SemaphoreType.DMA,
        pltpu.SemaphoreType.DMA,
        pltpu.SemaphoreType.DMA,
    ],
)
def _combine(pos0_hbm, pos1_hbm, w0x_hbm, w1x_hbm, y_hbm, out_hbm,
             i0_v, i1_v, w0x_v, w1x_v, ra_v, rb_v,
             semA, semB, semA2, semB2, semO):
    c = lax.axis_index("c")
    s = lax.axis_index("s")
    wid = s * NC + c
    base = wid * TPW
    H = TPW // 2
    pltpu.sync_copy(pos0_hbm.at[pl.ds(base, TPW)], i0_v)
    pltpu.sync_copy(pos1_hbm.at[pl.ds(base, TPW)], i1_v)
    cpA1 = pltpu.async_copy(y_hbm.at[i0_v.at[pl.ds(0, H)]],
                            ra_v.at[pl.ds(0, H)], semA)
    cpB1 = pltpu.async_copy(y_hbm.at[i1_v.at[pl.ds(0, H)]],
                            rb_v.at[pl.ds(0, H)], semB)
    cpA2 = pltpu.async_copy(y_hbm.at[i0_v.at[pl.ds(H, H)]],
                            ra_v.at[pl.ds(H, H)], semA2)
    cpB2 = pltpu.async_copy(y_hbm.at[i1_v.at[pl.ds(H, H)]],
                            rb_v.at[pl.ds(H, H)], semB2)
    pltpu.sync_copy(w0x_hbm.at[pl.ds(base, TPW)], w0x_v)
    pltpu.sync_copy(w1x_hbm.at[pl.ds(base, TPW)], w1x_v)
    cpA1.wait()
    cpB1.wait()

    def add_lo(j, _):
        sl = pl.ds(j * 16, 16)
        for r in range(H):
            wa = w0x_v[r, pl.ds(0, 16)]
            wb = w1x_v[r, pl.ds(0, 16)]
            ra_v[r, sl] = wa * ra_v[r, sl] + wb * rb_v[r, sl]
        return 0
    lax.fori_loop(0, D // 16, add_lo, 0)
    cpO = pltpu.async_copy(ra_v.at[pl.ds(0, H)],
                           out_hbm.at[pl.ds(base, H)], semO)
    cpA2.wait()
    cpB2.wait()

    def add_hi(j, _):
        sl = pl.ds(j * 16, 16)
        for r in range(H, TPW):
            wa = w0x_v[r, pl.ds(0, 16)]
            wb = w1x_v[r, pl.ds(0, 16)]
            ra_v[r, sl] = wa * ra_v[r, sl] + wb * rb_v[r, sl]
        return 0
    lax.fori_loop(0, D // 16, add_hi, 0)
    cpO.wait()
    pltpu.sync_copy(ra_v.at[pl.ds(H, H)], out_hbm.at[pl.ds(base + H, H)])


def kernel(hidden_states, use_grouped_topk, top_k, router_logits, renormalize, W13, W2):
    logits = router_logits.astype(jnp.float32)
    rn2d = jnp.broadcast_to(
        jnp.asarray(renormalize, jnp.float32)[None, None], (8, 128))
    pos0x, pos1x, w0x, w1x, be8 = _route(logits, rn2d)
    pos0 = pos0x.reshape(T)
    pos1 = pos1x.reshape(T)
    be = be8[0, :NBLK]
    xg = _dispatch(hidden_states, pos0, pos1)
    y = _gemm(be, xg, W13, W2)
    return _combine(pos0, pos1, w0x, w1x, y)


# speedup vs baseline: 1.1962x; 1.0277x over previous
import functools

import jax
import jax.numpy as jnp
from jax import lax
from jax.experimental import pallas as pl
from jax.experimental.pallas import tpu as pltpu
from jax.experimental.pallas import tpu_sc as plsc

E = 8
TOPK = 2
D = 768
DFF = 2048
T = 2048
B = 512
NBLK = -(-(TOPK * T) // B) + E
P = NBLK * B
NC = 2
NS = 16
NW = NC * NS
TPW = T // NW
NG = 4

_mesh = plsc.VectorSubcoreMesh(core_axis_name="c", subcore_axis_name="s")


def _route_body(lg_ref, rn_ref, pos0_ref, pos1_ref, w0_ref, w1_ref, be_ref):
    lane = lax.broadcasted_iota(jnp.int32, (T, 128), 1)
    lane8 = lax.broadcasted_iota(jnp.int32, (T, E), 1)
    lg = lg_ref[...]

    m1 = jnp.max(lg, axis=1, keepdims=True)
    e0 = jnp.min(jnp.where(lg == m1, lane8, 999), axis=1, keepdims=True)
    lg2 = jnp.where(lane8 == e0, jnp.float32(-3e38), lg)
    m2 = jnp.max(lg2, axis=1, keepdims=True)
    e1 = jnp.min(jnp.where(lg2 == m2, lane8, 999), axis=1, keepdims=True)

    d12 = jnp.exp(m2 - m1)
    w1n = 1.0 / (1.0 + d12)
    w2n = d12 * w1n
    den8 = jnp.sum(jnp.exp(lg - m1), axis=1, keepdims=True)
    p1 = 1.0 / den8
    p2 = d12 / den8
    rn = rn_ref[0:1, 0:1] > 0.5
    wa = jnp.where(rn, w1n, p1)
    wb = jnp.where(rn, w2n, p2)

    oh0 = (lane == e0).astype(jnp.float32)
    oh1 = (lane == e1).astype(jnp.float32)
    oh = oh0 + oh1

    r_i = lax.broadcasted_iota(jnp.int32, (T, T), 0)
    c_i = lax.broadcasted_iota(jnp.int32, (T, T), 1)
    ltri = (c_i < r_i).astype(jnp.float32)
    pre = lax.dot_general(ltri, oh, (((1,), (0,)), ((), ())),
                          preferred_element_type=jnp.float32)

    counts = pre[T - 1:T, :] + oh[T - 1:T, :]
    nblk = jnp.floor((counts + float(B - 1)) * (1.0 / B))
    u_r = lax.broadcasted_iota(jnp.int32, (128, 128), 0)
    u_c = lax.broadcasted_iota(jnp.int32, (128, 128), 1)
    utri = (u_r < u_c).astype(jnp.float32)
    blkbase = lax.dot_general(nblk, utri, (((1,), (0,)), ((), ())),
                              preferred_element_type=jnp.float32)
    rowbase = blkbase * float(B)

    slot = rowbase + pre
    pos0 = jnp.sum(jnp.where(lane == e0, slot, 0.0), axis=1, keepdims=True)
    pos1 = jnp.sum(jnp.where(lane == e1, slot, 0.0), axis=1, keepdims=True)
    pos0_ref[...] = jnp.reshape(pos0.astype(jnp.int32), (T // 128, 128))
    pos1_ref[...] = jnp.reshape(pos1.astype(jnp.int32), (T // 128, 128))
    w0_ref[...] = jnp.broadcast_to(wa, (T, 128))
    w1_ref[...] = jnp.broadcast_to(wb, (T, 128))

    bvec = lax.broadcasted_iota(jnp.int32, (8, 128), 1).astype(jnp.float32)
    becnt = jnp.zeros((8, 128), jnp.int32)
    for e in range(E):
        becnt = becnt + (bvec >= blkbase[0:1, e:e + 1]).astype(jnp.int32)
    be_ref[...] = jnp.maximum(becnt - 1, 0)


def _route(logits, rn2d):
    return pl.pallas_call(
        _route_body,
        grid=(1,),
        in_specs=[
            pl.BlockSpec((T, E), lambda i: (0, 0)),
            pl.BlockSpec((8, 128), lambda i: (0, 0)),
        ],
        out_specs=[
            pl.BlockSpec((T // 128, 128), lambda i: (0, 0)),
            pl.BlockSpec((T // 128, 128), lambda i: (0, 0)),
            pl.BlockSpec((T, 128), lambda i: (0, 0)),
            pl.BlockSpec((T, 128), lambda i: (0, 0)),
            pl.BlockSpec((8, 128), lambda i: (0, 0)),
        ],
        out_shape=[
            jax.ShapeDtypeStruct((T // 128, 128), jnp.int32),
            jax.ShapeDtypeStruct((T // 128, 128), jnp.int32),
            jax.ShapeDtypeStruct((T, 128), jnp.float32),
            jax.ShapeDtypeStruct((T, 128), jnp.float32),
            jax.ShapeDtypeStruct((8, 128), jnp.int32),
        ],
    )(logits, rn2d)


@functools.partial(
    pl.kernel,
    out_type=jax.ShapeDtypeStruct((P, D), jnp.float32),
    mesh=_mesh,
    scratch_types=[
        pltpu.VMEM((TPW,), jnp.int32),
        pltpu.VMEM((TPW,), jnp.int32),
        pltpu.VMEM((TPW, D), jnp.float32),
        pltpu.SemaphoreType.DMA,
        pltpu.SemaphoreType.DMA,
    ],
)
def _dispatch(x_hbm, pos0_hbm, pos1_hbm, xg_hbm,
              i0_v, i1_v, rows_v, semA, semB):
    c = lax.axis_index("c")
    s = lax.axis_index("s")
    wid = s * NC + c
    base = wid * TPW
    pltpu.sync_copy(x_hbm.at[pl.ds(base, TPW)], rows_v)
    pltpu.sync_copy(pos0_hbm.at[pl.ds(base, TPW)], i0_v)
    pltpu.sync_copy(pos1_hbm.at[pl.ds(base, TPW)], i1_v)
    cpA = pltpu.async_copy(rows_v, xg_hbm.at[i0_v], semA)
    cpB = pltpu.async_copy(rows_v, xg_hbm.at[i1_v], semB)
    cpA.wait()
    cpB.wait()


def _clampE(i):
    return jnp.minimum(jnp.maximum(i, 0), E - 1)


def _gemm_body(be_ref, x_ref, *refs):
    y_ref = refs[-1]
    ws = refs[:-1]
    ng = len(ws) // 3
    x = x_ref[...].astype(jnp.bfloat16)
    p = None
    for q in range(ng):
        gq = ws[q]
        uq = ws[ng + q]
        w2q = ws[2 * ng + q]
        h1 = lax.dot_general(x, gq[0][0].astype(jnp.bfloat16),
                             (((1,), (1,)), ((), ())),
                             preferred_element_type=jnp.float32)
        h2 = lax.dot_general(x, uq[0][0].astype(jnp.bfloat16),
                             (((1,), (1,)), ((), ())),
                             preferred_element_type=jnp.float32)
        a = (h1 * jax.nn.sigmoid(h1) * h2).astype(jnp.bfloat16)
        pq = lax.dot_general(a, w2q[0].astype(jnp.bfloat16),
                             (((1,), (1,)), ((), ())),
                             preferred_element_type=jnp.float32)
        p = pq if p is None else p + pq
    y_ref[...] = p


def _gemm(be, xg, W13, W2):
    grid_spec = pltpu.PrefetchScalarGridSpec(
        num_scalar_prefetch=1,
        grid=(NBLK,),
        in_specs=(
            [pl.BlockSpec((B, D), lambda b, be_ref: (b, 0))]
            + [pl.BlockSpec(
                   (1, 1, DFF // NG, D),
                   functools.partial(
                       lambda q, b, be_ref: (_clampE(be_ref[b]), q, 0, 0), q))
               for q in range(NG)]
            + [pl.BlockSpec(
                   (1, 1, DFF // NG, D),
                   functools.partial(
                       lambda q, b, be_ref: (_clampE(be_ref[b]), NG + q, 0, 0),
                       q))
               for q in range(NG)]
            + [pl.BlockSpec(
                   (1, D, DFF // NG),
                   functools.partial(
                       lambda q, b, be_ref: (_clampE(be_ref[b]), 0, q), q))
               for q in range(NG)]
        ),
        out_specs=pl.BlockSpec((B, D), lambda b, be_ref: (b, 0)),
    )
    w13_q = W13.reshape(E, 2 * NG, DFF // NG, D)
    return pl.pallas_call(
        _gemm_body,
        grid_spec=grid_spec,
        out_shape=jax.ShapeDtypeStruct((P, D), jnp.float32),
        compiler_params=pltpu.CompilerParams(
            dimension_semantics=("arbitrary",),
        ),
    )(be, xg, *([w13_q] * (2 * NG)), *([W2] * NG))


@functools.partial(
    pl.kernel,
    out_type=jax.ShapeDtypeStruct((T, D), jnp.float32),
    mesh=_mesh,
    scratch_types=[
        pltpu.VMEM((TPW,), jnp.int32),
        pltpu.VMEM((TPW,), jnp.int32),
        pltpu.VMEM((TPW, 128), jnp.float32),
        pltpu.VMEM((TPW, 128), jnp.float32),
        pltpu.VMEM((TPW, D), jnp.float32),
        pltpu.VMEM((TPW, D), jnp.float32),
        pltpu.SemaphoreType.DMA,
        pltpu.SemaphoreType.DMA,
        pltpu.SemaphoreType.DMA,
        pltpu.SemaphoreType.DMA,
        pltpu.SemaphoreType.DMA,
    ],
)
def _combine(pos0_hbm, pos1_hbm, w0x_hbm, w1x_hbm, y_hbm, out_hbm,
             i0_v, i1_v, w0x_v, w1x_v, ra_v, rb_v,
             semA, semB, semA2, semB2, semO):
    c = lax.axis_index("c")
    s = lax.axis_index("s")
    wid = s * NC + c
    base = wid * TPW
    H = TPW // 2
    pltpu.sync_copy(pos0_hbm.at[pl.ds(base, TPW)], i0_v)
    pltpu.sync_copy(pos1_hbm.at[pl.ds(base, TPW)], i1_v)
    cpA1 = pltpu.async_copy(y_hbm.at[i0_v.at[pl.ds(0, H)]],
                            ra_v.at[pl.ds(0, H)], semA)
    cpB1 = pltpu.async_copy(y_hbm.at[i1_v.at[pl.ds(0, H)]],
                            rb_v.at[pl.ds(0, H)], semB)
    cpA2 = pltpu.async_copy(y_hbm.at[i0_v.at[pl.ds(H, H)]],
                            ra_v.at[pl.ds(H, H)], semA2)
    cpB2 = pltpu.async_copy(y_hbm.at[i1_v.at[pl.ds(H, H)]],
                            rb_v.at[pl.ds(H, H)], semB2)
    pltpu.sync_copy(w0x_hbm.at[pl.ds(base, TPW)], w0x_v)
    pltpu.sync_copy(w1x_hbm.at[pl.ds(base, TPW)], w1x_v)
    cpA1.wait()
    cpB1.wait()

    def add_lo(j, _):
        sl = pl.ds(j * 16, 16)
        for r in range(H):
            wa = w0x_v[r, pl.ds(0, 16)]
            wb = w1x_v[r, pl.ds(0, 16)]
            ra_v[r, sl] = wa * ra_v[r, sl] + wb * rb_v[r, sl]
        return 0
    lax.fori_loop(0, D // 16, add_lo, 0)
    cpO = pltpu.async_copy(ra_v.at[pl.ds(0, H)],
                           out_hbm.at[pl.ds(base, H)], semO)
    cpA2.wait()
    cpB2.wait()

    def add_hi(j, _):
        sl = pl.ds(j * 16, 16)
        for r in range(H, TPW):
            wa = w0x_v[r, pl.ds(0, 16)]
            wb = w1x_v[r, pl.ds(0, 16)]
            ra_v[r, sl] = wa * ra_v[r, sl] + wb * rb_v[r, sl]
        return 0
    lax.fori_loop(0, D // 16, add_hi, 0)
    cpO.wait()
    pltpu.sync_copy(ra_v.at[pl.ds(H, H)], out_hbm.at[pl.ds(base + H, H)])


def kernel(hidden_states, use_grouped_topk, top_k, router_logits, renormalize, W13, W2):
    logits = router_logits.astype(jnp.float32)
    rn2d = jnp.broadcast_to(
        jnp.asarray(renormalize, jnp.float32)[None, None], (8, 128))
    pos0x, pos1x, w0x, w1x, be8 = _route(logits, rn2d)
    pos0 = pos0x.reshape(T)
    pos1 = pos1x.reshape(T)
    be = be8[0, :NBLK]
    xg = _dispatch(hidden_states, pos0, pos1)
    y = _gemm(be, xg, W13, W2)
    return _combine(pos0, pos1, w0x, w1x, y)


# speedup vs baseline: 1.1985x; 1.0019x over previous
import functools

import jax
import jax.numpy as jnp
from jax import lax
from jax.experimental import pallas as pl
from jax.experimental.pallas import tpu as pltpu
from jax.experimental.pallas import tpu_sc as plsc

E = 8
TOPK = 2
D = 768
DFF = 2048
T = 2048
B = 512
NBLK = -(-(TOPK * T) // B) + E
P = NBLK * B
NC = 2
NS = 16
NW = NC * NS
TPW = T // NW
NG = 4

_mesh = plsc.VectorSubcoreMesh(core_axis_name="c", subcore_axis_name="s")


def _route_body(lg_ref, rn_ref, pos0_ref, pos1_ref, w0_ref, w1_ref, be_ref):
    lane = lax.broadcasted_iota(jnp.int32, (T, 128), 1)
    lane8 = lax.broadcasted_iota(jnp.int32, (T, E), 1)
    lg = lg_ref[...]

    m1 = jnp.max(lg, axis=1, keepdims=True)
    e0 = jnp.min(jnp.where(lg == m1, lane8, 999), axis=1, keepdims=True)
    lg2 = jnp.where(lane8 == e0, jnp.float32(-3e38), lg)
    m2 = jnp.max(lg2, axis=1, keepdims=True)
    e1 = jnp.min(jnp.where(lg2 == m2, lane8, 999), axis=1, keepdims=True)

    d12 = jnp.exp(m2 - m1)
    w1n = 1.0 / (1.0 + d12)
    w2n = d12 * w1n
    den8 = jnp.sum(jnp.exp(lg - m1), axis=1, keepdims=True)
    p1 = 1.0 / den8
    p2 = d12 / den8
    rn = rn_ref[0:1, 0:1] > 0.5
    wa = jnp.where(rn, w1n, p1)
    wb = jnp.where(rn, w2n, p2)

    oh0 = (lane == e0).astype(jnp.float32)
    oh1 = (lane == e1).astype(jnp.float32)
    oh = oh0 + oh1

    r_i = lax.broadcasted_iota(jnp.int32, (T, T), 0)
    c_i = lax.broadcasted_iota(jnp.int32, (T, T), 1)
    ltri = (c_i < r_i).astype(jnp.float32)
    pre = lax.dot_general(ltri, oh, (((1,), (0,)), ((), ())),
                          preferred_element_type=jnp.float32)

    counts = pre[T - 1:T, :] + oh[T - 1:T, :]
    nblk = jnp.floor((counts + float(B - 1)) * (1.0 / B))
    u_r = lax.broadcasted_iota(jnp.int32, (128, 128), 0)
    u_c = lax.broadcasted_iota(jnp.int32, (128, 128), 1)
    utri = (u_r < u_c).astype(jnp.float32)
    blkbase = lax.dot_general(nblk, utri, (((1,), (0,)), ((), ())),
                              preferred_element_type=jnp.float32)
    rowbase = blkbase * float(B)

    slot = rowbase + pre
    pos0 = jnp.sum(jnp.where(lane == e0, slot, 0.0), axis=1, keepdims=True)
    pos1 = jnp.sum(jnp.where(lane == e1, slot, 0.0), axis=1, keepdims=True)
    pos0_ref[...] = jnp.reshape(pos0.astype(jnp.int32), (T // 128, 128))
    pos1_ref[...] = jnp.reshape(pos1.astype(jnp.int32), (T // 128, 128))
    w0_ref[...] = jnp.broadcast_to(wa, (T, 128))
    w1_ref[...] = jnp.broadcast_to(wb, (T, 128))

    bvec = lax.broadcasted_iota(jnp.int32, (8, 128), 1).astype(jnp.float32)
    becnt = jnp.zeros((8, 128), jnp.int32)
    for e in range(E):
        becnt = becnt + (bvec >= blkbase[0:1, e:e + 1]).astype(jnp.int32)
    be_ref[...] = jnp.maximum(becnt - 1, 0)


def _route(logits, rn2d):
    return pl.pallas_call(
        _route_body,
        grid=(1,),
        in_specs=[
            pl.BlockSpec((T, E), lambda i: (0, 0)),
            pl.BlockSpec((8, 128), lambda i: (0, 0)),
        ],
        out_specs=[
            pl.BlockSpec((T // 128, 128), lambda i: (0, 0)),
            pl.BlockSpec((T // 128, 128), lambda i: (0, 0)),
            pl.BlockSpec((T, 128), lambda i: (0, 0)),
            pl.BlockSpec((T, 128), lambda i: (0, 0)),
            pl.BlockSpec((8, 128), lambda i: (0, 0)),
        ],
        out_shape=[
            jax.ShapeDtypeStruct((T // 128, 128), jnp.int32),
            jax.ShapeDtypeStruct((T // 128, 128), jnp.int32),
            jax.ShapeDtypeStruct((T, 128), jnp.float32),
            jax.ShapeDtypeStruct((T, 128), jnp.float32),
            jax.ShapeDtypeStruct((8, 128), jnp.int32),
        ],
    )(logits, rn2d)


@functools.partial(
    pl.kernel,
    out_type=jax.ShapeDtypeStruct((P, D), jnp.float32),
    mesh=_mesh,
    scratch_types=[
        pltpu.VMEM((TPW,), jnp.int32),
        pltpu.VMEM((TPW,), jnp.int32),
        pltpu.VMEM((TPW, D), jnp.float32),
        pltpu.SemaphoreType.DMA,
        pltpu.SemaphoreType.DMA,
    ],
)
def _dispatch(x_hbm, pos0_hbm, pos1_hbm, xg_hbm,
              i0_v, i1_v, rows_v, semA, semB):
    c = lax.axis_index("c")
    s = lax.axis_index("s")
    wid = s * NC + c
    base = wid * TPW
    pltpu.sync_copy(x_hbm.at[pl.ds(base, TPW)], rows_v)
    pltpu.sync_copy(pos0_hbm.at[pl.ds(base, TPW)], i0_v)
    pltpu.sync_copy(pos1_hbm.at[pl.ds(base, TPW)], i1_v)
    cpA = pltpu.async_copy(rows_v, xg_hbm.at[i0_v], semA)
    cpB = pltpu.async_copy(rows_v, xg_hbm.at[i1_v], semB)
    cpA.wait()
    cpB.wait()


def _clampE(i):
    return jnp.minimum(jnp.maximum(i, 0), E - 1)


def _gemm_body(be_ref, x_ref, *refs):
    y_ref = refs[-1]
    ws = refs[:-1]
    ng = len(ws) // 3
    x = x_ref[...]
    p = None
    for q in range(ng):
        gq = ws[q]
        uq = ws[ng + q]
        w2q = ws[2 * ng + q]
        h1 = lax.dot_general(x, gq[0][0], (((1,), (1,)), ((), ())),
                             preferred_element_type=jnp.float32)
        h2 = lax.dot_general(x, uq[0][0], (((1,), (1,)), ((), ())),
                             preferred_element_type=jnp.float32)
        a = h1 * jax.nn.sigmoid(h1) * h2
        pq = lax.dot_general(a, w2q[0], (((1,), (1,)), ((), ())),
                             preferred_element_type=jnp.float32)
        p = pq if p is None else p + pq
    y_ref[...] = p


def _gemm(be, xg, W13, W2):
    grid_spec = pltpu.PrefetchScalarGridSpec(
        num_scalar_prefetch=1,
        grid=(NBLK,),
        in_specs=(
            [pl.BlockSpec((B, D), lambda b, be_ref: (b, 0))]
            + [pl.BlockSpec(
                   (1, 1, DFF // NG, D),
                   functools.partial(
                       lambda q, b, be_ref: (_clampE(be_ref[b]), q, 0, 0), q))
               for q in range(NG)]
            + [pl.BlockSpec(
                   (1, 1, DFF // NG, D),
                   functools.partial(
                       lambda q, b, be_ref: (_clampE(be_ref[b]), NG + q, 0, 0),
                       q))
               for q in range(NG)]
            + [pl.BlockSpec(
                   (1, D, DFF // NG),
                   functools.partial(
                       lambda q, b, be_ref: (_clampE(be_ref[b]), 0, q), q))
               for q in range(NG)]
        ),
        out_specs=pl.BlockSpec((B, D), lambda b, be_ref: (b, 0)),
    )
    w13_q = W13.reshape(E, 2 * NG, DFF // NG, D)
    return pl.pallas_call(
        _gemm_body,
        grid_spec=grid_spec,
        out_shape=jax.ShapeDtypeStruct((P, D), jnp.float32),
        compiler_params=pltpu.CompilerParams(
            dimension_semantics=("arbitrary",),
        ),
    )(be, xg, *([w13_q] * (2 * NG)), *([W2] * NG))


@functools.partial(
    pl.kernel,
    out_type=jax.ShapeDtypeStruct((T, D), jnp.float32),
    mesh=_mesh,
    scratch_types=[
        pltpu.VMEM((TPW,), jnp.int32),
        pltpu.VMEM((TPW,), jnp.int32),
        pltpu.VMEM((TPW, 128), jnp.float32),
        pltpu.VMEM((TPW, 128), jnp.float32),
        pltpu.VMEM((TPW, D), jnp.float32),
        pltpu.VMEM((TPW, D), jnp.float32),
        pltpu.SemaphoreType.DMA,
        pltpu.SemaphoreType.DMA,
        pltpu.SemaphoreType.DMA,
        pltpu.SemaphoreType.DMA,
        pltpu.SemaphoreType.DMA,
    ],
)
def _combine(pos0_hbm, pos1_hbm, w0x_hbm, w1x_hbm, y_hbm, out_hbm,
             i0_v, i1_v, w0x_v, w1x_v, ra_v, rb_v,
             semA, semB, semA2, semB2, semO):
    c = lax.axis_index("c")
    s = lax.axis_index("s")
    wid = s * NC + c
    base = wid * TPW
    H = TPW // 2
    pltpu.sync_copy(pos0_hbm.at[pl.ds(base, TPW)], i0_v)
    pltpu.sync_copy(pos1_hbm.at[pl.ds(base, TPW)], i1_v)
    cpA1 = pltpu.async_copy(y_hbm.at[i0_v.at[pl.ds(0, H)]],
                            ra_v.at[pl.ds(0, H)], semA)
    cpB1 = pltpu.async_copy(y_hbm.at[i1_v.at[pl.ds(0, H)]],
                            rb_v.at[pl.ds(0, H)], semB)
    cpA2 = pltpu.async_copy(y_hbm.at[i0_v.at[pl.ds(H, H)]],
                            ra_v.at[pl.ds(H, H)], semA2)
    cpB2 = pltpu.async_copy(y_hbm.at[i1_v.at[pl.ds(H, H)]],
                            rb_v.at[pl.ds(H, H)], semB2)
    pltpu.sync_copy(w0x_hbm.at[pl.ds(base, TPW)], w0x_v)
    pltpu.sync_copy(w1x_hbm.at[pl.ds(base, TPW)], w1x_v)
    cpA1.wait()
    cpB1.wait()

    def add_lo(j, _):
        sl = pl.ds(j * 16, 16)
        for r in range(H):
            wa = w0x_v[r, pl.ds(0, 16)]
            wb = w1x_v[r, pl.ds(0, 16)]
            ra_v[r, sl] = wa * ra_v[r, sl] + wb * rb_v[r, sl]
        return 0
    lax.fori_loop(0, D // 16, add_lo, 0)
    cpO = pltpu.async_copy(ra_v.at[pl.ds(0, H)],
                           out_hbm.at[pl.ds(base, H)], semO)
    cpA2.wait()
    cpB2.wait()

    def add_hi(j, _):
        sl = pl.ds(j * 16, 16)
        for r in range(H, TPW):
            wa = w0x_v[r, pl.ds(0, 16)]
            wb = w1x_v[r, pl.ds(0, 16)]
            ra_v[r, sl] = wa * ra_v[r, sl] + wb * rb_v[r, sl]
        return 0
    lax.fori_loop(0, D // 16, add_hi, 0)
    cpO.wait()
    pltpu.sync_copy(ra_v.at[pl.ds(H, H)], out_hbm.at[pl.ds(base + H, H)])


def kernel(hidden_states, use_grouped_topk, top_k, router_logits, renormalize, W13, W2):
    logits = router_logits.astype(jnp.float32)
    rn2d = jnp.broadcast_to(
        jnp.asarray(renormalize, jnp.float32)[None, None], (8, 128))
    pos0x, pos1x, w0x, w1x, be8 = _route(logits, rn2d)
    pos0 = pos0x.reshape(T)
    pos1 = pos1x.reshape(T)
    be = be8[0, :NBLK]
    xg = _dispatch(hidden_states, pos0, pos1)
    y = _gemm(be, xg, W13, W2)
    return _combine(pos0, pos1, w0x, w1x, y)
